# baseline (speedup 1.0000x reference)
import functools

import jax
import jax.numpy as jnp
from jax import lax
from jax.experimental import pallas as pl
from jax.experimental.pallas import tpu as pltpu
from jax.experimental.pallas import tpu_sc as plsc

VOCAB = 1000
VPAD = 1024
D_MODEL = 32
BATCH = 1024
SEQ = 50
SPAD = 56
IDX_STRIDE = 64

NC, NS = 2, 16
NW = NC * NS
B_PER_W = BATCH // NW
N_PAIR = B_PER_W // 2
IDX_PER_W = B_PER_W * IDX_STRIDE


def _proj_body(emb_ref, w_ref, b_ref, p_ref):
    p_ref[...] = lax.dot_general(
        emb_ref[...], w_ref[...], (((1,), (1,)), ((), ())),
        preferred_element_type=jnp.float32) + b_ref[...]


def _compute_table(emb, W, b):
    w_pad = jnp.pad(W, ((0, VPAD - VOCAB), (0, 0)))
    b_pad = jnp.pad(b, (0, VPAD - VOCAB)).reshape(1, VPAD)
    return pl.pallas_call(
        _proj_body,
        out_shape=jax.ShapeDtypeStruct((VOCAB, VPAD), jnp.float32),
    )(emb, w_pad, b_pad)


@functools.cache
def _make_gather_kernel():
    mesh = plsc.VectorSubcoreMesh(core_axis_name="c", subcore_axis_name="s",
                                  num_cores=NC, num_subcores=NS)

    @functools.partial(
        pl.kernel,
        out_type=jax.ShapeDtypeStruct((BATCH, SEQ, VOCAB), jnp.float32),
        mesh=mesh,
        scratch_types=[
            pltpu.VMEM((IDX_PER_W,), jnp.int32),
            pltpu.VMEM((SPAD, VPAD), jnp.float32),
            pltpu.VMEM((SPAD, VPAD), jnp.float32),
            pltpu.SemaphoreType.DMA,
            pltpu.SemaphoreType.DMA,
        ],
    )
    def gather_kernel(p_hbm, xf_hbm, out_hbm, idx_v, buf0, buf1, sem0, sem1):
        wid = lax.axis_index("s") * NC + lax.axis_index("c")
        base = wid * B_PER_W
        pltpu.sync_copy(xf_hbm.at[pl.ds(base * IDX_STRIDE, IDX_PER_W)], idx_v)

        def _start(bi, buf, sem):
            pltpu.async_copy(
                p_hbm.at[idx_v.at[pl.ds(bi * IDX_STRIDE, SPAD)]], buf, sem)

        def _finish(bi, buf, sem):
            pltpu.make_async_copy(
                p_hbm.at[idx_v.at[pl.ds(bi * IDX_STRIDE, SPAD)]],
                buf, sem).wait()
            pltpu.sync_copy(
                buf, out_hbm.at[base + bi].at[pl.ds(0, SPAD), pl.ds(0, VPAD)])

        _start(0, buf0, sem0)

        def _pair(i, carry):
            b0 = 2 * i
            _start(b0 + 1, buf1, sem1)
            _finish(b0, buf0, sem0)

            @pl.when(b0 + 2 < B_PER_W)
            def _():
                _start(b0 + 2, buf0, sem0)

            _finish(b0 + 1, buf1, sem1)
            return carry

        lax.fori_loop(0, N_PAIR, _pair, 0)

    return gather_kernel


def kernel(x, emb, W, b):
    P = _compute_table(emb, W, b)
    xf = jnp.pad(x, ((0, 0), (0, IDX_STRIDE - SEQ)), mode="wrap").reshape(-1)
    return _make_gather_kernel()(P, xf)


# speedup vs baseline: 3.1748x; 3.1748x over previous
import functools

import jax
import jax.numpy as jnp
from jax import lax
from jax.experimental import pallas as pl
from jax.experimental.pallas import tpu as pltpu
from jax.experimental.pallas import tpu_sc as plsc

VOCAB = 1000
D_MODEL = 32
BATCH = 1024
SEQ = 50
LANES = 16
N_VEC = BATCH // LANES

NC, NS = 2, 16
NW = NC * NS


@functools.cache
def _make_lookup_kernel():
    mesh = plsc.VectorSubcoreMesh(core_axis_name="c", subcore_axis_name="s",
                                  num_cores=NC, num_subcores=NS)

    @functools.partial(
        pl.kernel,
        out_type=jax.ShapeDtypeStruct((SEQ, D_MODEL, BATCH), jnp.float32),
        mesh=mesh,
        compiler_params=pltpu.CompilerParams(use_tc_tiling_on_sc=False,
                                             needs_layout_passes=False),
        scratch_types=[
            pltpu.VMEM((D_MODEL, BATCH), jnp.float32),
            pltpu.VMEM((BATCH,), jnp.int32),
            pltpu.VMEM((D_MODEL, BATCH), jnp.float32),
        ],
    )
    def lookup_kernel(embt_hbm, xt_hbm, ht_hbm, tab_v, idx_v, h_v):
        wid = lax.axis_index("s") * NC + lax.axis_index("c")
        pltpu.sync_copy(embt_hbm, tab_v)

        def _one_l(l):
            pltpu.sync_copy(xt_hbm.at[l], idx_v)

            def _col_group(g, carry):
                cols = idx_v[pl.ds(g * LANES, LANES)]

                def _row(d, inner):
                    rows = jnp.full((LANES,), d, dtype=jnp.int32)
                    h_v[d, pl.ds(g * LANES, LANES)] = plsc.load_gather(
                        tab_v, [rows, cols])
                    return inner

                lax.fori_loop(0, D_MODEL, _row, 0)
                return carry

            lax.fori_loop(0, N_VEC, _col_group, 0)
            pltpu.sync_copy(h_v, ht_hbm.at[l])

        _one_l(wid)

        @pl.when(wid + NW < SEQ)
        def _():
            _one_l(wid + NW)

    return lookup_kernel


def _proj_body(w_ref, b_ref, h_ref, o_ref):
    o_ref[0] = lax.dot_general(
        w_ref[...], h_ref[0], (((1,), (0,)), ((), ())),
        preferred_element_type=jnp.float32) + b_ref[...]


def _project(W, b, h_t):
    return pl.pallas_call(
        _proj_body,
        grid=(SEQ,),
        in_specs=[
            pl.BlockSpec((VOCAB, D_MODEL), lambda l: (0, 0)),
            pl.BlockSpec((VOCAB, 1), lambda l: (0, 0)),
            pl.BlockSpec((1, D_MODEL, BATCH), lambda l: (l, 0, 0)),
        ],
        out_specs=pl.BlockSpec((1, VOCAB, BATCH), lambda l: (l, 0, 0)),
        out_shape=jax.ShapeDtypeStruct((SEQ, VOCAB, BATCH), jnp.float32),
    )(W, b.reshape(VOCAB, 1), h_t)


def kernel(x, emb, W, b):
    x_t = x.T
    emb_t = jnp.pad(emb.T, ((0, 0), (0, BATCH - VOCAB)))
    h_t = _make_lookup_kernel()(emb_t, x_t)
    out_t = _project(W, b, h_t)
    return jnp.transpose(out_t, (2, 0, 1))


# speedup vs baseline: 3.4043x; 1.0723x over previous
import functools

import jax
import jax.numpy as jnp
from jax import lax
from jax.experimental import pallas as pl
from jax.experimental.pallas import tpu as pltpu
from jax.experimental.pallas import tpu_sc as plsc

VOCAB = 1000
D_MODEL = 32
BATCH = 1024
SEQ = 50
LANES = 16
N_VEC = BATCH // LANES

NC, NS = 2, 16
NW = NC * NS


@functools.cache
def _make_lookup_kernel():
    mesh = plsc.VectorSubcoreMesh(core_axis_name="c", subcore_axis_name="s",
                                  num_cores=NC, num_subcores=NS)

    @functools.partial(
        pl.kernel,
        out_type=jax.ShapeDtypeStruct((SEQ, D_MODEL, BATCH), jnp.float32),
        mesh=mesh,
        compiler_params=pltpu.CompilerParams(needs_layout_passes=False),
        scratch_types=[
            pltpu.VMEM((D_MODEL, BATCH), jnp.float32),
            pltpu.VMEM((BATCH,), jnp.int32),
            pltpu.VMEM((D_MODEL, BATCH), jnp.float32),
        ],
    )
    def lookup_kernel(embt_hbm, xt_hbm, ht_hbm, tab_v, idx_v, h_v):
        wid = lax.axis_index("s") * NC + lax.axis_index("c")
        pltpu.sync_copy(embt_hbm, tab_v)

        def _one_l(l):
            pltpu.sync_copy(xt_hbm.at[l], idx_v)

            def _col_group(g, carry):
                cols = idx_v[pl.ds(g * LANES, LANES)]
                for d in range(D_MODEL):
                    rows = jnp.full((LANES,), d, dtype=jnp.int32)
                    h_v[d, pl.ds(g * LANES, LANES)] = plsc.load_gather(
                        tab_v, [rows, cols])
                return carry

            lax.fori_loop(0, N_VEC, _col_group, 0)
            pltpu.sync_copy(h_v, ht_hbm.at[l])

        _one_l(wid)

        @pl.when(wid + NW < SEQ)
        def _():
            _one_l(wid + NW)

    return lookup_kernel


def _proj_body(w_ref, b_ref, h_ref, o_ref):
    o_ref[0] = lax.dot_general(
        w_ref[...], h_ref[0], (((1,), (0,)), ((), ())),
        preferred_element_type=jnp.float32) + b_ref[...]


def _project(W, b, h_t):
    return pl.pallas_call(
        _proj_body,
        grid=(SEQ,),
        in_specs=[
            pl.BlockSpec((VOCAB, D_MODEL), lambda l: (0, 0)),
            pl.BlockSpec((VOCAB, 1), lambda l: (0, 0)),
            pl.BlockSpec((1, D_MODEL, BATCH), lambda l: (l, 0, 0)),
        ],
        out_specs=pl.BlockSpec((1, VOCAB, BATCH), lambda l: (l, 0, 0)),
        out_shape=jax.ShapeDtypeStruct((SEQ, VOCAB, BATCH), jnp.float32),
    )(W, b.reshape(VOCAB, 1), h_t)


def kernel(x, emb, W, b):
    x_t = x.T
    emb_t = jnp.pad(emb.T, ((0, 0), (0, BATCH - VOCAB)))
    h_t = _make_lookup_kernel()(emb_t, x_t)
    out_t = _project(W, b, h_t)
    return jnp.transpose(out_t, (2, 0, 1))
